# baseline (speedup 1.0000x reference)
import functools

import jax
import jax.numpy as jnp
from jax import lax
from jax.experimental import pallas as pl
from jax.experimental.pallas import tpu as pltpu
from jax.experimental.pallas import tpu_sc as plsc

NUM_FIELDS = 26
VOCAB = 100000
EMB = 50
B = 4096
NUM_NUM = 13

NW = 32
ROWS = B * NUM_FIELDS
ROWS_PER_W = ROWS // NW
GRP = 128
GRPS_PER_W = ROWS_PER_W // GRP
CHUNK_GRPS = 13
N_CHUNKS = GRPS_PER_W // CHUNK_GRPS
CHUNK_ROWS = CHUNK_GRPS * GRP


def _sc_gather(tab_flat, idx3d):
    mesh = plsc.VectorSubcoreMesh(core_axis_name="c", subcore_axis_name="s")

    @functools.partial(
        pl.kernel,
        out_type=jax.ShapeDtypeStruct((ROWS, EMB), jnp.float32),
        mesh=mesh,
        scratch_types=[
            pltpu.VMEM((GRPS_PER_W, GRP), jnp.int32),
            pltpu.VMEM((CHUNK_ROWS, EMB), jnp.float32),
            pltpu.SemaphoreType.DMA,
        ],
        compiler_params=pltpu.CompilerParams(use_tc_tiling_on_sc=False),
    )
    def k(tab_hbm, idx_hbm, out_hbm, idx_v, rows_v, sem):
        wid = lax.axis_index("s") * 2 + lax.axis_index("c")
        g_base = wid * GRPS_PER_W

        pltpu.sync_copy(idx_hbm.at[wid], idx_v)
        for c in range(N_CHUNKS):
            handles = []
            for j in range(CHUNK_GRPS):
                handles.append(
                    pltpu.async_copy(
                        tab_hbm.at[idx_v.at[c * CHUNK_GRPS + j]],
                        rows_v.at[pl.ds(j * GRP, GRP)],
                        sem,
                    )
                )
            for h in handles:
                h.wait()
            pltpu.sync_copy(
                rows_v,
                out_hbm.at[pl.ds((g_base + c * CHUNK_GRPS) * GRP, CHUNK_ROWS)],
            )

    return k(tab_flat, idx3d)


def _mlp_body(xn_ref, xe_ref, w1n_ref, w1e_ref, b1_ref, w2_ref, b2_ref,
              w3_ref, b3_ref, out_ref):
    h = (
        jnp.dot(xn_ref[...], w1n_ref[...], preferred_element_type=jnp.float32)
        + jnp.dot(xe_ref[...], w1e_ref[...], preferred_element_type=jnp.float32)
        + b1_ref[...]
    )
    h = jnp.maximum(h, 0.0)
    h = jnp.dot(h, w2_ref[...], preferred_element_type=jnp.float32) + b2_ref[...]
    h = jnp.maximum(h, 0.0)
    out_ref[...] = (
        jnp.dot(h, w3_ref[...], preferred_element_type=jnp.float32) + b3_ref[...]
    )


def _tc_mlp(x_num, x_emb, W1n, W1e, b1, W2, b2, W3, b3):
    BLK = 512
    grid = (B // BLK,)
    e_dim = x_emb.shape[1]
    return pl.pallas_call(
        _mlp_body,
        grid=grid,
        in_specs=[
            pl.BlockSpec((BLK, NUM_NUM), lambda i: (i, 0)),
            pl.BlockSpec((BLK, e_dim), lambda i: (i, 0)),
            pl.BlockSpec((NUM_NUM, 128), lambda i: (0, 0)),
            pl.BlockSpec((e_dim, 128), lambda i: (0, 0)),
            pl.BlockSpec((1, 128), lambda i: (0, 0)),
            pl.BlockSpec((128, 64), lambda i: (0, 0)),
            pl.BlockSpec((1, 64), lambda i: (0, 0)),
            pl.BlockSpec((64, 1), lambda i: (0, 0)),
            pl.BlockSpec((1, 1), lambda i: (0, 0)),
        ],
        out_specs=pl.BlockSpec((BLK, 1), lambda i: (i, 0)),
        out_shape=jax.ShapeDtypeStruct((B, 1), jnp.float32),
    )(x_num, x_emb, W1n, W1e, b1, W2, b2, W3, b3)


def kernel(x_num, x_cat, tables, W1, b1, W2, b2, W3, b3):
    tab_flat = tables.reshape(NUM_FIELDS * VOCAB, EMB)
    flat_idx = (
        x_cat + (jnp.arange(NUM_FIELDS, dtype=jnp.int32) * VOCAB)[None, :]
    ).reshape(ROWS)
    rows = jnp.take(tab_flat, flat_idx, axis=0)
    x_emb = rows.reshape(B, NUM_FIELDS * EMB)
    out = _tc_mlp(
        x_num, x_emb,
        W1[:NUM_NUM], W1[NUM_NUM:],
        b1.reshape(1, 128), W2, b2.reshape(1, 64), W3, b3.reshape(1, 1),
    )
    return out.reshape(B)


# speedup vs baseline: 7.7842x; 7.7842x over previous
import functools

import jax
import jax.numpy as jnp
from jax import lax
from jax.experimental import pallas as pl
from jax.experimental.pallas import tpu as pltpu
from jax.experimental.pallas import tpu_sc as plsc

NUM_FIELDS = 26
VOCAB = 100000
EMB = 50
B = 4096
NUM_NUM = 13

NW = 32
ROWS = B * NUM_FIELDS
ROWS_PER_W = ROWS // NW
CHUNK_ROWS = 416
N_CHUNKS = ROWS_PER_W // CHUNK_ROWS
GRP = 128
GRPS_PER_CHUNK = CHUNK_ROWS * 4 // GRP
TAB16_ROWS = NUM_FIELDS * VOCAB * EMB // 16
WIN = 64


def _sc_gather(tab16, idx4):
    mesh = plsc.VectorSubcoreMesh(core_axis_name="c", subcore_axis_name="s")

    @functools.partial(
        pl.kernel,
        out_type=jax.ShapeDtypeStruct((4 * ROWS, 16), jnp.float32),
        mesh=mesh,
        scratch_types=[
            pltpu.VMEM((GRPS_PER_CHUNK, GRP), jnp.int32),
            pltpu.VMEM((GRPS_PER_CHUNK * GRP, 16), jnp.float32),
            pltpu.SemaphoreType.DMA,
        ],
        compiler_params=pltpu.CompilerParams(use_tc_tiling_on_sc=False),
    )
    def k(tab_hbm, idx_hbm, out_hbm, idx_v, st_v, sem):
        wid = lax.axis_index("s") * 2 + lax.axis_index("c")

        def chunk_body(c, carry):
            pltpu.sync_copy(idx_hbm.at[wid, c], idx_v)
            hs = []
            for j in range(GRPS_PER_CHUNK):
                hs.append(
                    pltpu.async_copy(
                        tab_hbm.at[idx_v.at[j]],
                        st_v.at[pl.ds(j * GRP, GRP)],
                        sem,
                    )
                )
            for h in hs:
                h.wait()
            out0 = (wid * ROWS_PER_W + c * CHUNK_ROWS) * 4
            pltpu.sync_copy(st_v, out_hbm.at[pl.ds(out0, 4 * CHUNK_ROWS)])
            return carry

        lax.fori_loop(0, N_CHUNKS, chunk_body, 0)

    return k(tab16, idx4)


def _mlp_body(xn_ref, st_ref, sh_ref, w1n_ref, w1e_ref, b1_ref, w2_ref,
              b2_ref, w3_ref, b3_ref, out_ref):
    h = jnp.dot(xn_ref[...], w1n_ref[...], preferred_element_type=jnp.float32)
    sh = sh_ref[...]
    for f in range(NUM_FIELDS):
        x = st_ref[:, f * WIN:(f + 1) * WIN]
        s = sh[:, f:f + 1]
        for k in range(3):
            amt = 2 << k
            rolled = jnp.concatenate([x[:, amt:], x[:, :amt]], axis=1)
            x = jnp.where((s & (1 << k)) != 0, rolled, x)
        h = h + jnp.dot(x, w1e_ref[pl.ds(f * WIN, WIN), :],
                        preferred_element_type=jnp.float32)
    h = jnp.maximum(h + b1_ref[...], 0.0)
    h = jnp.maximum(
        jnp.dot(h, w2_ref[...], preferred_element_type=jnp.float32)
        + b2_ref[...], 0.0)
    out_ref[...] = (
        jnp.dot(h, w3_ref[...], preferred_element_type=jnp.float32)
        + b3_ref[...]
    )


def _tc_mlp(x_num, staged, shift8, W1n, W1e_pad, b1, W2, b2, W3, b3):
    BLK = 512
    grid = (B // BLK,)
    K = NUM_FIELDS * WIN
    return pl.pallas_call(
        _mlp_body,
        grid=grid,
        in_specs=[
            pl.BlockSpec((BLK, NUM_NUM), lambda i: (i, 0)),
            pl.BlockSpec((BLK, K), lambda i: (i, 0)),
            pl.BlockSpec((BLK, NUM_FIELDS), lambda i: (i, 0)),
            pl.BlockSpec((NUM_NUM, 128), lambda i: (0, 0)),
            pl.BlockSpec((K, 128), lambda i: (0, 0)),
            pl.BlockSpec((1, 128), lambda i: (0, 0)),
            pl.BlockSpec((128, 64), lambda i: (0, 0)),
            pl.BlockSpec((1, 64), lambda i: (0, 0)),
            pl.BlockSpec((64, 1), lambda i: (0, 0)),
            pl.BlockSpec((1, 1), lambda i: (0, 0)),
        ],
        out_specs=pl.BlockSpec((BLK, 1), lambda i: (i, 0)),
        out_shape=jax.ShapeDtypeStruct((B, 1), jnp.float32),
    )(x_num, staged, shift8, W1n, W1e_pad, b1, W2, b2, W3, b3)


def kernel(x_num, x_cat, tables, W1, b1, W2, b2, W3, b3):
    tab16 = tables.reshape(TAB16_ROWS, 16)
    flat_idx = x_cat + (jnp.arange(NUM_FIELDS, dtype=jnp.int32) * VOCAB)[None, :]
    base16 = (flat_idx * EMB) // 16
    idx4 = (
        base16.reshape(ROWS, 1) + jnp.arange(4, dtype=jnp.int32)[None, :]
    ).reshape(NW, N_CHUNKS, GRPS_PER_CHUNK, GRP)
    shift8 = flat_idx % 8

    staged = _sc_gather(tab16, idx4)
    staged2d = staged.reshape(B, NUM_FIELDS * WIN)

    W1e_pad = jnp.pad(
        W1[NUM_NUM:].reshape(NUM_FIELDS, EMB, 128),
        ((0, 0), (0, WIN - EMB), (0, 0)),
    ).reshape(NUM_FIELDS * WIN, 128)

    out = _tc_mlp(
        x_num, staged2d, shift8,
        W1[:NUM_NUM], W1e_pad,
        b1.reshape(1, 128), W2, b2.reshape(1, 64), W3, b3.reshape(1, 1),
    )
    return out.reshape(B)


# speedup vs baseline: 28.3989x; 3.6483x over previous
import functools

import jax
import jax.numpy as jnp
from jax import lax
from jax.experimental import pallas as pl
from jax.experimental.pallas import tpu as pltpu
from jax.experimental.pallas import tpu_sc as plsc

NUM_FIELDS = 26
VOCAB = 100000
EMB = 50
B = 4096
NUM_NUM = 13

NW = 32
ROWS = B * NUM_FIELDS
ROWS_PER_W = ROWS // NW
CHUNK = 416
N_CHUNKS = ROWS_PER_W // CHUNK
GRP = 104
GRPS = CHUNK // GRP
WIN = 64

VBLK = 512
DEPAD_GRID = -(-VOCAB // VBLK)
VROWS = DEPAD_GRID * (VBLK // 2)


def _depad_body(in_ref, out_ref):
    H = VBLK // 2
    pad = jnp.zeros((NUM_FIELDS, H, WIN - EMB), jnp.float32)
    x = in_ref[...]
    ya = jnp.transpose(x[:, :, :H], (0, 2, 1))
    yb = jnp.transpose(x[:, :, H:], (0, 2, 1))
    out_ref[...] = jnp.concatenate([ya, pad, yb, pad], axis=2)


def _tc_depad(t2):
    return pl.pallas_call(
        _depad_body,
        grid=(DEPAD_GRID,),
        in_specs=[pl.BlockSpec((NUM_FIELDS, EMB, VBLK), lambda i: (0, 0, i))],
        out_specs=pl.BlockSpec((NUM_FIELDS, VBLK // 2, 2 * WIN), lambda i: (0, i, 0)),
        out_shape=jax.ShapeDtypeStruct((NUM_FIELDS, VROWS, 2 * WIN), jnp.float32),
    )(t2)


def _sc_gather(tab64, idx):
    mesh = plsc.VectorSubcoreMesh(core_axis_name="c", subcore_axis_name="s")

    @functools.partial(
        pl.kernel,
        out_type=jax.ShapeDtypeStruct((ROWS, WIN), jnp.float32),
        mesh=mesh,
        scratch_types=[
            pltpu.VMEM((GRPS, GRP), jnp.int32),
            pltpu.VMEM((CHUNK, WIN), jnp.float32),
            pltpu.SemaphoreType.DMA,
        ],
        compiler_params=pltpu.CompilerParams(use_tc_tiling_on_sc=False),
    )
    def k(tab_hbm, idx_hbm, out_hbm, idx_v, st_v, sem):
        wid = lax.axis_index("s") * 2 + lax.axis_index("c")

        def chunk_body(c, carry):
            pltpu.sync_copy(idx_hbm.at[wid, c], idx_v)
            hs = []
            for j in range(GRPS):
                hs.append(
                    pltpu.async_copy(
                        tab_hbm.at[idx_v.at[j]],
                        st_v.at[pl.ds(j * GRP, GRP)],
                        sem,
                    )
                )
            for h in hs:
                h.wait()
            r0 = wid * ROWS_PER_W + c * CHUNK
            pltpu.sync_copy(st_v, out_hbm.at[pl.ds(r0, CHUNK)])
            return carry

        lax.fori_loop(0, N_CHUNKS, chunk_body, 0)

    return k(tab64, idx)


def _mlp_body(xn_ref, st_ref, w1n_ref, w1e_ref, b1_ref, w2_ref, b2_ref,
              w3_ref, b3_ref, out_ref):
    h = (
        jnp.dot(xn_ref[...], w1n_ref[...], preferred_element_type=jnp.float32)
        + jnp.dot(st_ref[...], w1e_ref[...], preferred_element_type=jnp.float32)
        + b1_ref[...]
    )
    h = jnp.maximum(h, 0.0)
    h = jnp.maximum(
        jnp.dot(h, w2_ref[...], preferred_element_type=jnp.float32)
        + b2_ref[...], 0.0)
    out_ref[...] = (
        jnp.dot(h, w3_ref[...], preferred_element_type=jnp.float32)
        + b3_ref[...]
    )


def _tc_mlp(x_num, staged, W1n, W1e_pad, b1, W2, b2, W3, b3):
    BLK = 512
    K = NUM_FIELDS * WIN
    return pl.pallas_call(
        _mlp_body,
        grid=(B // BLK,),
        in_specs=[
            pl.BlockSpec((BLK, NUM_NUM), lambda i: (i, 0)),
            pl.BlockSpec((BLK, K), lambda i: (i, 0)),
            pl.BlockSpec((NUM_NUM, 128), lambda i: (0, 0)),
            pl.BlockSpec((K, 128), lambda i: (0, 0)),
            pl.BlockSpec((1, 128), lambda i: (0, 0)),
            pl.BlockSpec((128, 64), lambda i: (0, 0)),
            pl.BlockSpec((1, 64), lambda i: (0, 0)),
            pl.BlockSpec((64, 1), lambda i: (0, 0)),
            pl.BlockSpec((1, 1), lambda i: (0, 0)),
        ],
        out_specs=pl.BlockSpec((BLK, 1), lambda i: (i, 0)),
        out_shape=jax.ShapeDtypeStruct((B, 1), jnp.float32),
    )(x_num, staged, W1n, W1e_pad, b1, W2, b2, W3, b3)


def kernel(x_num, x_cat, tables, W1, b1, W2, b2, W3, b3):
    t2 = jnp.transpose(tables, (0, 2, 1))
    T = _tc_depad(t2)
    tab64 = T.reshape(NUM_FIELDS * VROWS * 2, WIN)

    H = VBLK // 2
    f_off = (jnp.arange(NUM_FIELDS, dtype=jnp.int32) * VROWS)[None, :]
    row_in_f = H * (x_cat // VBLK) + (x_cat % H)
    flat_idx = (
        (f_off + row_in_f) * 2 + (x_cat % VBLK) // H
    ).reshape(NW, N_CHUNKS, GRPS, GRP)

    staged = _sc_gather(tab64, flat_idx)
    staged2d = staged.reshape(B, NUM_FIELDS * WIN)

    W1e_pad = jnp.pad(
        W1[NUM_NUM:].reshape(NUM_FIELDS, EMB, 128),
        ((0, 0), (0, WIN - EMB), (0, 0)),
    ).reshape(NUM_FIELDS * WIN, 128)

    out = _tc_mlp(
        x_num, staged2d,
        W1[:NUM_NUM], W1e_pad,
        b1.reshape(1, 128), W2, b2.reshape(1, 64), W3, b3.reshape(1, 1),
    )
    return out.reshape(B)


# speedup vs baseline: 30.6945x; 1.0808x over previous
import functools

import jax
import jax.numpy as jnp
from jax import lax
from jax.experimental import pallas as pl
from jax.experimental.pallas import tpu as pltpu
from jax.experimental.pallas import tpu_sc as plsc

NUM_FIELDS = 26
VOCAB = 100000
EMB = 50
B = 4096
NUM_NUM = 13

NW = 32
ROWS = B * NUM_FIELDS
ROWS_PER_W = ROWS // NW
CHUNK = 416
N_CHUNKS = ROWS_PER_W // CHUNK
GRP = 104
GRPS = CHUNK // GRP
WIN = 64

VBLK = 1024
DEPAD_GRID = -(-VOCAB // VBLK)
VROWS = DEPAD_GRID * (VBLK // 2)


def _depad_body(in_ref, out_ref):
    H = VBLK // 2
    pad = jnp.zeros((NUM_FIELDS, H, WIN - EMB), jnp.float32)
    x = in_ref[...]
    eye = jnp.eye(EMB, dtype=jnp.float32)
    ya = jax.lax.dot_general(
        x[:, :, :H], eye, (((1,), (0,)), ((), ())),
        preferred_element_type=jnp.float32)
    yb = jax.lax.dot_general(
        x[:, :, H:], eye, (((1,), (0,)), ((), ())),
        preferred_element_type=jnp.float32)
    out_ref[...] = jnp.concatenate([ya, pad, yb, pad], axis=2)


def _tc_depad(t2):
    return pl.pallas_call(
        _depad_body,
        grid=(DEPAD_GRID,),
        in_specs=[pl.BlockSpec((NUM_FIELDS, EMB, VBLK), lambda i: (0, 0, i))],
        out_specs=pl.BlockSpec((NUM_FIELDS, VBLK // 2, 2 * WIN), lambda i: (0, i, 0)),
        out_shape=jax.ShapeDtypeStruct((NUM_FIELDS, VROWS, 2 * WIN), jnp.float32),
    )(t2)


def _sc_gather(tab64, idx):
    mesh = plsc.VectorSubcoreMesh(core_axis_name="c", subcore_axis_name="s")

    @functools.partial(
        pl.kernel,
        out_type=jax.ShapeDtypeStruct((ROWS, WIN), jnp.float32),
        mesh=mesh,
        scratch_types=[
            pltpu.VMEM((GRPS, GRP), jnp.int32),
            pltpu.VMEM((CHUNK, WIN), jnp.float32),
            pltpu.SemaphoreType.DMA,
        ],
        compiler_params=pltpu.CompilerParams(use_tc_tiling_on_sc=False),
    )
    def k(tab_hbm, idx_hbm, out_hbm, idx_v, st_v, sem):
        wid = lax.axis_index("s") * 2 + lax.axis_index("c")

        def chunk_body(c, carry):
            pltpu.sync_copy(idx_hbm.at[wid, c], idx_v)
            hs = []
            for j in range(GRPS):
                hs.append(
                    pltpu.async_copy(
                        tab_hbm.at[idx_v.at[j]],
                        st_v.at[pl.ds(j * GRP, GRP)],
                        sem,
                    )
                )
            for h in hs:
                h.wait()
            r0 = wid * ROWS_PER_W + c * CHUNK
            pltpu.sync_copy(st_v, out_hbm.at[pl.ds(r0, CHUNK)])
            return carry

        lax.fori_loop(0, N_CHUNKS, chunk_body, 0)

    return k(tab64, idx)


def _mlp_body(xn_ref, st_ref, w1n_ref, w1e_ref, b1_ref, w2_ref, b2_ref,
              w3_ref, b3_ref, out_ref):
    h = (
        jnp.dot(xn_ref[...], w1n_ref[...], preferred_element_type=jnp.float32)
        + jnp.dot(st_ref[...], w1e_ref[...], preferred_element_type=jnp.float32)
        + b1_ref[...]
    )
    h = jnp.maximum(h, 0.0)
    h = jnp.maximum(
        jnp.dot(h, w2_ref[...], preferred_element_type=jnp.float32)
        + b2_ref[...], 0.0)
    out_ref[...] = (
        jnp.dot(h, w3_ref[...], preferred_element_type=jnp.float32)
        + b3_ref[...]
    )


def _tc_mlp(x_num, staged, W1n, W1e_pad, b1, W2, b2, W3, b3):
    BLK = 512
    K = NUM_FIELDS * WIN
    return pl.pallas_call(
        _mlp_body,
        grid=(B // BLK,),
        in_specs=[
            pl.BlockSpec((BLK, NUM_NUM), lambda i: (i, 0)),
            pl.BlockSpec((BLK, K), lambda i: (i, 0)),
            pl.BlockSpec((NUM_NUM, 128), lambda i: (0, 0)),
            pl.BlockSpec((K, 128), lambda i: (0, 0)),
            pl.BlockSpec((1, 128), lambda i: (0, 0)),
            pl.BlockSpec((128, 64), lambda i: (0, 0)),
            pl.BlockSpec((1, 64), lambda i: (0, 0)),
            pl.BlockSpec((64, 1), lambda i: (0, 0)),
            pl.BlockSpec((1, 1), lambda i: (0, 0)),
        ],
        out_specs=pl.BlockSpec((BLK, 1), lambda i: (i, 0)),
        out_shape=jax.ShapeDtypeStruct((B, 1), jnp.float32),
    )(x_num, staged, W1n, W1e_pad, b1, W2, b2, W3, b3)


def kernel(x_num, x_cat, tables, W1, b1, W2, b2, W3, b3):
    t2 = jnp.transpose(tables, (0, 2, 1))
    T = _tc_depad(t2)
    tab64 = T.reshape(NUM_FIELDS * VROWS * 2, WIN)

    H = VBLK // 2
    f_off = (jnp.arange(NUM_FIELDS, dtype=jnp.int32) * VROWS)[None, :]
    row_in_f = H * (x_cat // VBLK) + (x_cat % H)
    flat_idx = (
        (f_off + row_in_f) * 2 + (x_cat % VBLK) // H
    ).reshape(NW, N_CHUNKS, GRPS, GRP)

    staged = _sc_gather(tab64, flat_idx)
    staged2d = staged.reshape(B, NUM_FIELDS * WIN)

    W1e_pad = jnp.pad(
        W1[NUM_NUM:].reshape(NUM_FIELDS, EMB, 128),
        ((0, 0), (0, WIN - EMB), (0, 0)),
    ).reshape(NUM_FIELDS * WIN, 128)

    out = _tc_mlp(
        x_num, staged2d,
        W1[:NUM_NUM], W1e_pad,
        b1.reshape(1, 128), W2, b2.reshape(1, 64), W3, b3.reshape(1, 1),
    )
    return out.reshape(B)
